# baseline (speedup 1.0000x reference)
import functools
import math

import jax
import jax.numpy as jnp
from jax import lax
from jax.experimental import pallas as pl
from jax.experimental.pallas import tpu as pltpu
from jax.experimental.pallas import tpu_sc as plsc

_HEADS = 8

_INV2PI = 0.15915494309189535
_COS_COEFFS = (
    0.9999999999193508, -19.739208758208584, 64.93939011340913,
    -85.45668538180254, 60.24246470872289, -26.406761080377983,
    7.806608463960106, -1.4609479689305238,
)


def _fast_cos(x):
    t = x * _INV2PI
    t = t - jnp.floor(t + 0.5)
    y = t * t
    acc = jnp.full_like(y, _COS_COEFFS[-1])
    for c in _COS_COEFFS[-2::-1]:
        acc = acc * y + c
    return acc


def _sc_gather(x, nbr_flat, batch):
    (R,) = nbr_flat.shape
    (Bb,) = batch.shape
    _, F = x.shape
    try:
        info = plsc.get_sparse_core_info()
        NW = info.num_cores * info.num_subcores
    except Exception:
        NW = 32
    CH = 128
    rpw = R // NW
    nch = rpw // CH
    bpw = Bb // NW
    assert rpw % CH == 0 and bpw <= CH

    nbr2d = nbr_flat.reshape(R // CH, CH)
    mesh = plsc.VectorSubcoreMesh(core_axis_name="c", subcore_axis_name="s")

    @functools.partial(
        pl.kernel,
        mesh=mesh,
        out_type=[
            jax.ShapeDtypeStruct((R, F), x.dtype),
            jax.ShapeDtypeStruct((Bb, F), x.dtype),
        ],
        scratch_types=[
            pltpu.VMEM((nch, CH), jnp.int32),
            pltpu.VMEM((2, CH, F), x.dtype),
            pltpu.VMEM((bpw,), jnp.int32),
            pltpu.VMEM((bpw, F), x.dtype),
            pltpu.SemaphoreType.DMA,
            pltpu.SemaphoreType.DMA,
            pltpu.SemaphoreType.DMA,
        ],
    )
    def k(x_hbm, nidx_hbm, bidx_hbm, g_out, gs_out,
          idx_v, buf_v, bidx_v, bbuf_v, sem0, sem1, semb):
        wid = lax.axis_index("s") * 2 + lax.axis_index("c")
        base = wid * rpw
        pltpu.sync_copy(nidx_hbm.at[pl.ds(wid * nch, nch)], idx_v)
        pltpu.sync_copy(bidx_hbm.at[pl.ds(wid * bpw, bpw)], bidx_v)
        bcp = pltpu.async_copy(x_hbm.at[bidx_v], bbuf_v, semb)

        sems = (sem0, sem1)
        cp0 = pltpu.async_copy(x_hbm.at[idx_v.at[0]], buf_v.at[0], sems[0])

        def body(g, _):
            for b in range(2):
                c = g * 2 + b
                nxt = (b + 1) % 2

                @pl.when(c + 1 < nch)
                def _():
                    pltpu.async_copy(
                        x_hbm.at[idx_v.at[c + 1]], buf_v.at[nxt], sems[nxt])

                pltpu.make_async_copy(
                    x_hbm.at[idx_v.at[c]], buf_v.at[b], sems[b]).wait()
                pltpu.sync_copy(
                    buf_v.at[b], g_out.at[pl.ds(base + c * CH, CH)])
            return 0

        lax.fori_loop(0, nch // 2, body, 0, unroll=False)
        bcp.wait()
        pltpu.sync_copy(bbuf_v, gs_out.at[pl.ds(wid * bpw, bpw)])

    return k(x, nbr2d, batch)


def _tc_body(BB, S, D,
             g_ref, gs_ref, tsf_ref, rel_ref,
             Win, bin_, tw, tb, sp,
             Wqh, Wqtr, Wkh, Wktr, Wvh, Wvtr,
             Wo, Wm1s, Wm1v, bm1, Wm2, bm2, Wout, bout,
             e8, e8t, out_ref):
    BS = BB * S
    f32 = jnp.float32
    dot = lambda a, b: lax.dot(a, b, preferred_element_type=f32)

    proj = jnp.maximum(dot(g_ref[:], Win[:]) + bin_[:], 0.0)
    src_x = jnp.maximum(dot(gs_ref[:], Win[:]) + bin_[:], 0.0)

    tsf = tsf_ref[:]
    mx = jnp.max(tsf.reshape(BB, S, 1), axis=1, keepdims=True)
    mxf = jnp.broadcast_to(mx, (BB, S, 1)).reshape(BS, 1)
    tsn = mxf - tsf
    tf = _fast_cos(tsn * tw[:] + tb[:])

    rel = rel_ref[:].reshape(BS, rel_ref.shape[-1])
    tr = jnp.concatenate([tf, rel], axis=1)
    K = dot(proj, Wkh[:]) + dot(tr, Wktr[:])
    V = dot(proj, Wvh[:]) + dot(tr, Wvtr[:])

    qtr = jnp.concatenate([_fast_cos(tb[:]), sp[:]], axis=1)
    q = dot(src_x, Wqh[:]) + dot(qtr, Wqtr[:])
    qexp = jnp.broadcast_to(q[:, None, :], (BB, S, D)).reshape(BS, D)

    scores = dot(K * qexp, e8[:])
    e = jnp.exp(scores)
    denom = jnp.sum(e.reshape(BB, S, _HEADS), axis=1)
    wexp = dot(e, e8t[:])
    val = jnp.sum((wexp * V).reshape(BB, S, D), axis=1)
    val = val / dot(denom, e8t[:])
    val = dot(val, Wo[:])

    mer = jnp.maximum(dot(src_x, Wm1s[:]) + dot(val, Wm1v[:]) + bm1[:], 0.0)
    mer = dot(mer, Wm2[:]) + bm2[:]
    out_ref[:] = dot(mer, Wout[:]) + bout[:]


def kernel(x, batch, nbr_idx, nbr_ts, nbr_rels, W_in, b_in, t_w, t_b,
           Wq, Wk, Wv, Wo, Wm1, bm1, Wm2, bm2, Wout, bout, src_param):
    N, F = x.shape
    B, S = nbr_idx.shape
    EF = nbr_rels.shape[-1]
    TF = t_w.shape[0]
    H = W_in.shape[1]
    OUT = Wout.shape[1]
    D = H + TF + EF
    dh = D // _HEADS
    R = B * S

    head_id = jnp.arange(D, dtype=jnp.int32) // dh
    ind = (head_id[:, None] == jnp.arange(_HEADS, dtype=jnp.int32)[None, :])
    e8 = ind.astype(jnp.float32)
    e8t = ind.T.astype(jnp.float32)

    xb = x
    Wqs = Wq * (1.0 / math.sqrt(float(dh)))
    weights = (
        W_in, b_in.reshape(1, H), t_w.reshape(1, TF),
        t_b.reshape(1, TF), src_param,
        Wqs[:H], Wqs[H:],
        Wk[:H], Wk[H:],
        Wv[:H], Wv[H:],
        Wo, Wm1[:H], Wm1[H:], bm1.reshape(1, H),
        Wm2, bm2.reshape(1, H), Wout, bout.reshape(1, OUT),
        e8, e8t,
    )

    BB = 128
    BS = BB * S

    blk = lambda shp, im: pl.BlockSpec(shp, im)
    row = lambda r, c: blk((r, c), lambda i: (i, 0))
    full = lambda r, c: blk((r, c), lambda i: (0, 0))

    def tc_call(G, Gs, ts_sub, rel_sub, Bsub):
        grid = (Bsub // BB,)
        return pl.pallas_call(
            functools.partial(_tc_body, BB, S, D),
            grid=grid,
            in_specs=[
                row(BS, F),
                row(BB, F),
                row(BS, 1),
                pl.BlockSpec((BB, S, EF), lambda i: (i, 0, 0)),
                full(F, H), full(1, H),
                full(1, TF), full(1, TF),
                full(1, EF),
                full(H, D), full(TF + EF, D),
                full(H, D), full(TF + EF, D),
                full(H, D), full(TF + EF, D),
                full(D, D),
                full(H, H), full(D, H),
                full(1, H),
                full(H, H), full(1, H),
                full(H, OUT), full(1, OUT),
                full(D, _HEADS), full(_HEADS, D),
            ],
            out_specs=row(BB, OUT),
            out_shape=jax.ShapeDtypeStruct((Bsub, OUT), jnp.float32),
        )(
            G, Gs, ts_sub.reshape(Bsub * S, 1),
            rel_sub,
            *weights,
        )

    NSPLIT = 1
    Bsub = B // NSPLIT
    outs = []
    for i in range(NSPLIT):
        sl = slice(i * Bsub, (i + 1) * Bsub)
        G, Gs = _sc_gather(xb, nbr_idx[sl].reshape(Bsub * S), batch[sl])
        outs.append(tc_call(G, Gs, nbr_ts[sl], nbr_rels[sl], Bsub))
    return jnp.concatenate(outs, axis=0)


# speedup vs baseline: 1.1765x; 1.1765x over previous
import functools
import math

import jax
import jax.numpy as jnp
from jax import lax
from jax.experimental import pallas as pl
from jax.experimental.pallas import tpu as pltpu
from jax.experimental.pallas import tpu_sc as plsc

_HEADS = 8

_INV2PI = 0.15915494309189535
_COS_COEFFS = (
    0.9999999999193508, -19.739208758208584, 64.93939011340913,
    -85.45668538180254, 60.24246470872289, -26.406761080377983,
    7.806608463960106, -1.4609479689305238,
)


def _fast_cos(x):
    t = x * _INV2PI
    t = t - jnp.floor(t + 0.5)
    y = t * t
    acc = jnp.full_like(y, _COS_COEFFS[-1])
    for c in _COS_COEFFS[-2::-1]:
        acc = acc * y + c
    return acc


def _sc_gather(x, nbr_flat, batch):
    (R,) = nbr_flat.shape
    (Bb,) = batch.shape
    _, F = x.shape
    try:
        info = plsc.get_sparse_core_info()
        NW = info.num_cores * info.num_subcores
    except Exception:
        NW = 32
    CH = 128
    rpw = R // NW
    nch = rpw // CH
    bpw = Bb // NW
    assert rpw % CH == 0 and bpw <= CH

    nbr2d = nbr_flat.reshape(R // CH, CH)
    mesh = plsc.VectorSubcoreMesh(core_axis_name="c", subcore_axis_name="s")

    @functools.partial(
        pl.kernel,
        mesh=mesh,
        out_type=[
            jax.ShapeDtypeStruct((R, F), x.dtype),
            jax.ShapeDtypeStruct((Bb, F), x.dtype),
        ],
        scratch_types=[
            pltpu.VMEM((nch, CH), jnp.int32),
            pltpu.VMEM((4, CH, F), x.dtype),
            pltpu.VMEM((bpw,), jnp.int32),
            pltpu.VMEM((bpw, F), x.dtype),
            [pltpu.SemaphoreType.DMA] * 4,
            [pltpu.SemaphoreType.DMA] * 4,
            pltpu.SemaphoreType.DMA,
        ],
    )
    def k(x_hbm, nidx_hbm, bidx_hbm, g_out, gs_out,
          idx_v, buf_v, bidx_v, bbuf_v, gsem, ssem, semb):
        NBUF = 4
        wid = lax.axis_index("s") * 2 + lax.axis_index("c")
        base = wid * rpw
        pltpu.sync_copy(nidx_hbm.at[pl.ds(wid * nch, nch)], idx_v)
        pltpu.sync_copy(bidx_hbm.at[pl.ds(wid * bpw, bpw)], bidx_v)
        bcp = pltpu.async_copy(x_hbm.at[bidx_v], bbuf_v, semb)

        for j in range(NBUF - 1):
            pltpu.async_copy(x_hbm.at[idx_v.at[j]], buf_v.at[j], gsem[j])

        def body(g, _):
            for b in range(NBUF):
                c = g * NBUF + b
                jn = (b + NBUF - 1) % NBUF

                @pl.when((c + NBUF - 1 < nch) & (c >= 1))
                def _():
                    pltpu.make_async_copy(
                        buf_v.at[jn], g_out.at[pl.ds(0, CH)], ssem[jn]).wait()

                @pl.when(c + NBUF - 1 < nch)
                def _():
                    pltpu.async_copy(
                        x_hbm.at[idx_v.at[c + NBUF - 1]], buf_v.at[jn],
                        gsem[jn])

                pltpu.make_async_copy(
                    x_hbm.at[idx_v.at[c]], buf_v.at[b], gsem[b]).wait()
                pltpu.async_copy(
                    buf_v.at[b], g_out.at[pl.ds(base + c * CH, CH)], ssem[b])
            return 0

        lax.fori_loop(0, nch // NBUF, body, 0, unroll=False)
        for j in range(NBUF):
            pltpu.make_async_copy(
                buf_v.at[j], g_out.at[pl.ds(0, CH)], ssem[j]).wait()
        bcp.wait()
        pltpu.sync_copy(bbuf_v, gs_out.at[pl.ds(wid * bpw, bpw)])

    return k(x, nbr2d, batch)


def _tc_body(BB, S, D,
             g_ref, gs_ref, tsf_ref, rel_ref,
             Win, bin_, tw, tb, sp,
             Wqh, Wqtr, Wkh, Wktr, Wvh, Wvtr,
             Wo, Wm1s, Wm1v, bm1, Wm2, bm2, Wout, bout,
             e8, e8t, out_ref):
    BS = BB * S
    f32 = jnp.float32
    dot = lambda a, b: lax.dot(a, b, preferred_element_type=f32)

    proj = jnp.maximum(dot(g_ref[:], Win[:]) + bin_[:], 0.0)
    src_x = jnp.maximum(dot(gs_ref[:], Win[:]) + bin_[:], 0.0)

    tsf = tsf_ref[:]
    mx = jnp.max(tsf.reshape(BB, S, 1), axis=1, keepdims=True)
    mxf = jnp.broadcast_to(mx, (BB, S, 1)).reshape(BS, 1)
    tsn = mxf - tsf
    tf = _fast_cos(tsn * tw[:] + tb[:])

    tr = jnp.concatenate([tf, rel_ref[:]], axis=1)
    K = dot(proj, Wkh[:]) + dot(tr, Wktr[:])
    V = dot(proj, Wvh[:]) + dot(tr, Wvtr[:])

    qtr = jnp.concatenate([_fast_cos(tb[:]), sp[:]], axis=1)
    q = dot(src_x, Wqh[:]) + dot(qtr, Wqtr[:])
    qexp = jnp.broadcast_to(q[:, None, :], (BB, S, D)).reshape(BS, D)

    scores = dot(K * qexp, e8[:])
    e = jnp.exp(scores)
    denom = jnp.sum(e.reshape(BB, S, _HEADS), axis=1)
    wexp = dot(e, e8t[:])
    val = jnp.sum((wexp * V).reshape(BB, S, D), axis=1)
    val = val / dot(denom, e8t[:])
    val = dot(val, Wo[:])

    mer = jnp.maximum(dot(src_x, Wm1s[:]) + dot(val, Wm1v[:]) + bm1[:], 0.0)
    mer = dot(mer, Wm2[:]) + bm2[:]
    out_ref[:] = dot(mer, Wout[:]) + bout[:]


def kernel(x, batch, nbr_idx, nbr_ts, nbr_rels, W_in, b_in, t_w, t_b,
           Wq, Wk, Wv, Wo, Wm1, bm1, Wm2, bm2, Wout, bout, src_param):
    N, F = x.shape
    B, S = nbr_idx.shape
    EF = nbr_rels.shape[-1]
    TF = t_w.shape[0]
    H = W_in.shape[1]
    OUT = Wout.shape[1]
    D = H + TF + EF
    dh = D // _HEADS
    R = B * S

    head_id = jnp.arange(D, dtype=jnp.int32) // dh
    ind = (head_id[:, None] == jnp.arange(_HEADS, dtype=jnp.int32)[None, :])
    e8 = ind.astype(jnp.float32)
    e8t = ind.T.astype(jnp.float32)

    xb = x
    Wqs = Wq * (1.0 / math.sqrt(float(dh)))
    weights = (
        W_in, b_in.reshape(1, H), t_w.reshape(1, TF),
        t_b.reshape(1, TF), src_param,
        Wqs[:H], Wqs[H:],
        Wk[:H], Wk[H:],
        Wv[:H], Wv[H:],
        Wo, Wm1[:H], Wm1[H:], bm1.reshape(1, H),
        Wm2, bm2.reshape(1, H), Wout, bout.reshape(1, OUT),
        e8, e8t,
    )

    BB = 128
    BS = BB * S

    blk = lambda shp, im: pl.BlockSpec(shp, im)
    row = lambda r, c: blk((r, c), lambda i: (i, 0))
    full = lambda r, c: blk((r, c), lambda i: (0, 0))

    def tc_call(G, Gs, ts_sub, rel_sub, Bsub):
        grid = (Bsub // BB,)
        return pl.pallas_call(
            functools.partial(_tc_body, BB, S, D),
            grid=grid,
            in_specs=[
                row(BS, F),
                row(BB, F),
                row(BS, 1),
                row(BS, EF),
                full(F, H), full(1, H),
                full(1, TF), full(1, TF),
                full(1, EF),
                full(H, D), full(TF + EF, D),
                full(H, D), full(TF + EF, D),
                full(H, D), full(TF + EF, D),
                full(D, D),
                full(H, H), full(D, H),
                full(1, H),
                full(H, H), full(1, H),
                full(H, OUT), full(1, OUT),
                full(D, _HEADS), full(_HEADS, D),
            ],
            out_specs=row(BB, OUT),
            out_shape=jax.ShapeDtypeStruct((Bsub, OUT), jnp.float32),
        )(
            G, Gs, ts_sub.reshape(Bsub * S, 1),
            rel_sub.reshape(Bsub * S, EF),
            *weights,
        )

    NSPLIT = 1
    Bsub = B // NSPLIT
    outs = []
    for i in range(NSPLIT):
        sl = slice(i * Bsub, (i + 1) * Bsub)
        G, Gs = _sc_gather(xb, nbr_idx[sl].reshape(Bsub * S), batch[sl])
        outs.append(tc_call(G, Gs, nbr_ts[sl], nbr_rels[sl], Bsub))
    return jnp.concatenate(outs, axis=0)


# speedup vs baseline: 1.2029x; 1.0225x over previous
import functools
import math

import jax
import jax.numpy as jnp
from jax import lax
from jax.experimental import pallas as pl
from jax.experimental.pallas import tpu as pltpu
from jax.experimental.pallas import tpu_sc as plsc

_HEADS = 8

_INV2PI = 0.15915494309189535
_COS_COEFFS = (
    0.9999994436793983, -19.739034372931126, 64.93061336990448,
    -85.29597096153826, 58.912555324414804, -21.28302159300549,
)


def _fast_cos(x):
    t = x * _INV2PI
    t = t - jnp.floor(t + 0.5)
    y = t * t
    acc = jnp.full_like(y, _COS_COEFFS[-1])
    for c in _COS_COEFFS[-2::-1]:
        acc = acc * y + c
    return acc


def _sc_gather(x, nbr_flat, batch):
    (R,) = nbr_flat.shape
    (Bb,) = batch.shape
    _, F = x.shape
    try:
        info = plsc.get_sparse_core_info()
        NW = info.num_cores * info.num_subcores
    except Exception:
        NW = 32
    CH = 128
    rpw = R // NW
    nch = rpw // CH
    bpw = Bb // NW
    assert rpw % CH == 0 and bpw <= CH

    nbr2d = nbr_flat.reshape(R // CH, CH)
    mesh = plsc.VectorSubcoreMesh(core_axis_name="c", subcore_axis_name="s")

    @functools.partial(
        pl.kernel,
        mesh=mesh,
        out_type=[
            jax.ShapeDtypeStruct((R, F), x.dtype),
            jax.ShapeDtypeStruct((Bb, F), x.dtype),
        ],
        scratch_types=[
            pltpu.VMEM((nch, CH), jnp.int32),
            pltpu.VMEM((4, CH, F), x.dtype),
            pltpu.VMEM((bpw,), jnp.int32),
            pltpu.VMEM((bpw, F), x.dtype),
            [pltpu.SemaphoreType.DMA] * 4,
            [pltpu.SemaphoreType.DMA] * 4,
            pltpu.SemaphoreType.DMA,
        ],
    )
    def k(x_hbm, nidx_hbm, bidx_hbm, g_out, gs_out,
          idx_v, buf_v, bidx_v, bbuf_v, gsem, ssem, semb):
        NBUF = 4
        wid = lax.axis_index("s") * 2 + lax.axis_index("c")
        base = wid * rpw
        pltpu.sync_copy(nidx_hbm.at[pl.ds(wid * nch, nch)], idx_v)
        pltpu.sync_copy(bidx_hbm.at[pl.ds(wid * bpw, bpw)], bidx_v)
        bcp = pltpu.async_copy(x_hbm.at[bidx_v], bbuf_v, semb)

        for j in range(NBUF - 1):
            pltpu.async_copy(x_hbm.at[idx_v.at[j]], buf_v.at[j], gsem[j])

        def body(g, _):
            for b in range(NBUF):
                c = g * NBUF + b
                jn = (b + NBUF - 1) % NBUF

                @pl.when((c + NBUF - 1 < nch) & (c >= 1))
                def _():
                    pltpu.make_async_copy(
                        buf_v.at[jn], g_out.at[pl.ds(0, CH)], ssem[jn]).wait()

                @pl.when(c + NBUF - 1 < nch)
                def _():
                    pltpu.async_copy(
                        x_hbm.at[idx_v.at[c + NBUF - 1]], buf_v.at[jn],
                        gsem[jn])

                pltpu.make_async_copy(
                    x_hbm.at[idx_v.at[c]], buf_v.at[b], gsem[b]).wait()
                pltpu.async_copy(
                    buf_v.at[b], g_out.at[pl.ds(base + c * CH, CH)], ssem[b])
            return 0

        lax.fori_loop(0, nch // NBUF, body, 0, unroll=False)
        for j in range(NBUF):
            pltpu.make_async_copy(
                buf_v.at[j], g_out.at[pl.ds(0, CH)], ssem[j]).wait()
        bcp.wait()
        pltpu.sync_copy(bbuf_v, gs_out.at[pl.ds(wid * bpw, bpw)])

    return k(x, nbr2d, batch)


def _tc_body(BB, S, D,
             g_ref, gs_ref, ts_ref, rel_ref,
             Win, bin_, tw, tb, sp,
             Wqh, Wqtr, Wkh, Wktr, Wvh, Wvtr,
             Wo, Wm1s, Wm1v, bm1, Wm2, bm2, Wout, bout,
             e8, e8t, out_ref):
    BS = BB * S
    f32 = jnp.float32
    dot = lambda a, b: lax.dot(a, b, preferred_element_type=f32)

    proj = jnp.maximum(dot(g_ref[:], Win[:]) + bin_[:], 0.0)
    src_x = jnp.maximum(dot(gs_ref[:], Win[:]) + bin_[:], 0.0)

    tsf = ts_ref[:]
    mx = jnp.max(tsf.reshape(BB, S, 1), axis=1, keepdims=True)
    mxf = jnp.broadcast_to(mx, (BB, S, 1)).reshape(BS, 1)
    tsn = mxf - tsf
    tf = _fast_cos(tsn * tw[:] + tb[:])

    tr = jnp.concatenate([tf, rel_ref[:]], axis=1)
    K = dot(proj, Wkh[:]) + dot(tr, Wktr[:])
    V = dot(proj, Wvh[:]) + dot(tr, Wvtr[:])

    qtr = jnp.concatenate([_fast_cos(tb[:]), sp[:]], axis=1)
    q = dot(src_x, Wqh[:]) + dot(qtr, Wqtr[:])
    qexp = jnp.broadcast_to(q[:, None, :], (BB, S, D)).reshape(BS, D)

    scores = dot(K * qexp, e8[:])
    e = jnp.exp(scores)
    denom = jnp.sum(e.reshape(BB, S, _HEADS), axis=1)
    wexp = dot(e, e8t[:])
    val = jnp.sum((wexp * V).reshape(BB, S, D), axis=1)
    val = val / dot(denom, e8t[:])
    val = dot(val, Wo[:])

    mer = jnp.maximum(dot(src_x, Wm1s[:]) + dot(val, Wm1v[:]) + bm1[:], 0.0)
    mer = dot(mer, Wm2[:]) + bm2[:]
    out_ref[:] = dot(mer, Wout[:]) + bout[:]


def kernel(x, batch, nbr_idx, nbr_ts, nbr_rels, W_in, b_in, t_w, t_b,
           Wq, Wk, Wv, Wo, Wm1, bm1, Wm2, bm2, Wout, bout, src_param):
    N, F = x.shape
    B, S = nbr_idx.shape
    EF = nbr_rels.shape[-1]
    TF = t_w.shape[0]
    H = W_in.shape[1]
    OUT = Wout.shape[1]
    D = H + TF + EF
    dh = D // _HEADS
    R = B * S

    head_id = jnp.arange(D, dtype=jnp.int32) // dh
    ind = (head_id[:, None] == jnp.arange(_HEADS, dtype=jnp.int32)[None, :])
    e8 = ind.astype(jnp.float32)
    e8t = ind.T.astype(jnp.float32)

    xb = x
    Wqs = Wq * (1.0 / math.sqrt(float(dh)))
    weights = (
        W_in, b_in.reshape(1, H), t_w.reshape(1, TF),
        t_b.reshape(1, TF), src_param,
        Wqs[:H], Wqs[H:],
        Wk[:H], Wk[H:],
        Wv[:H], Wv[H:],
        Wo, Wm1[:H], Wm1[H:], bm1.reshape(1, H),
        Wm2, bm2.reshape(1, H), Wout, bout.reshape(1, OUT),
        e8, e8t,
    )

    BB = 128
    BS = BB * S

    blk = lambda shp, im: pl.BlockSpec(shp, im)
    row = lambda r, c: blk((r, c), lambda i: (i, 0))
    full = lambda r, c: blk((r, c), lambda i: (0, 0))

    def tc_call(G, Gs, ts_sub, rel_sub, Bsub):
        grid = (Bsub // BB,)
        return pl.pallas_call(
            functools.partial(_tc_body, BB, S, D),
            grid=grid,
            in_specs=[
                row(BS, F),
                row(BB, F),
                row(BS, 1),
                row(BS, EF),
                full(F, H), full(1, H),
                full(1, TF), full(1, TF),
                full(1, EF),
                full(H, D), full(TF + EF, D),
                full(H, D), full(TF + EF, D),
                full(H, D), full(TF + EF, D),
                full(D, D),
                full(H, H), full(D, H),
                full(1, H),
                full(H, H), full(1, H),
                full(H, OUT), full(1, OUT),
                full(D, _HEADS), full(_HEADS, D),
            ],
            out_specs=row(BB, OUT),
            out_shape=jax.ShapeDtypeStruct((Bsub, OUT), jnp.float32),
        )(
            G, Gs, ts_sub.reshape(Bsub * S, 1),
            rel_sub.reshape(Bsub * S, EF),
            *weights,
        )

    NSPLIT = 1
    Bsub = B // NSPLIT
    outs = []
    for i in range(NSPLIT):
        sl = slice(i * Bsub, (i + 1) * Bsub)
        G, Gs = _sc_gather(xb, nbr_idx[sl].reshape(Bsub * S), batch[sl])
        outs.append(tc_call(G, Gs, nbr_ts[sl], nbr_rels[sl], Bsub))
    return jnp.concatenate(outs, axis=0)
